# baseline (speedup 1.0000x reference)
import functools

import jax
import jax.numpy as jnp
from jax.experimental import pallas as pl

N = 4096
D = 512
BM = 256


def _body(adj_ref, x_ref, w_ref, o_ref):
    a16 = adj_ref[...].astype(jnp.bfloat16)
    y = jnp.dot(a16, x_ref[...], preferred_element_type=jnp.float32)
    z = jnp.dot(y.astype(jnp.bfloat16), w_ref[...], preferred_element_type=jnp.float32)
    acc = jnp.maximum(z[:, 0:D], 0.0)
    acc = acc + jnp.maximum(z[:, D:2 * D], 0.0)
    acc = acc + jnp.maximum(z[:, 2 * D:3 * D], 0.0)
    acc = acc + jnp.maximum(z[:, 3 * D:4 * D], 0.0)
    o_ref[...] = acc


@jax.jit
def _run(x, adj, wcat):
    grid = (N // BM,)
    return pl.pallas_call(
        _body,
        grid=grid,
        in_specs=[
            pl.BlockSpec((BM, N), lambda i: (i, 0)),
            pl.BlockSpec((N, D), lambda i: (0, 0)),
            pl.BlockSpec((D, 4 * D), lambda i: (0, 0)),
        ],
        out_specs=pl.BlockSpec((BM, D), lambda i: (i, 0)),
        out_shape=jax.ShapeDtypeStruct((N, D), jnp.float32),
    )(adj, x, wcat)


def kernel(x, adj, now_epoch, W4, W8, W16, W32):
    wcat = jnp.concatenate([W4, W8, W16, W32], axis=1).astype(jnp.bfloat16)
    return _run(x.astype(jnp.bfloat16), adj, wcat)


# speedup vs baseline: 1.1814x; 1.1814x over previous
import functools

import jax
import jax.numpy as jnp
from jax.experimental import pallas as pl

N = 4096
D = 512
BM = 512


def _body(adj_ref, x_ref, w_ref, o_ref):
    y = jnp.dot(adj_ref[...], x_ref[...], preferred_element_type=jnp.float32)
    z = jnp.dot(y, w_ref[...], preferred_element_type=jnp.float32)
    acc = jnp.maximum(z[:, 0:D], 0.0)
    acc = acc + jnp.maximum(z[:, D:2 * D], 0.0)
    acc = acc + jnp.maximum(z[:, 2 * D:3 * D], 0.0)
    acc = acc + jnp.maximum(z[:, 3 * D:4 * D], 0.0)
    o_ref[...] = acc


@jax.jit
def _run(x, adj, wcat):
    grid = (N // BM,)
    return pl.pallas_call(
        _body,
        grid=grid,
        in_specs=[
            pl.BlockSpec((BM, N), lambda i: (i, 0)),
            pl.BlockSpec((N, D), lambda i: (0, 0)),
            pl.BlockSpec((D, 4 * D), lambda i: (0, 0)),
        ],
        out_specs=pl.BlockSpec((BM, D), lambda i: (i, 0)),
        out_shape=jax.ShapeDtypeStruct((N, D), jnp.float32),
    )(adj, x, wcat)


def kernel(x, adj, now_epoch, W4, W8, W16, W32):
    wcat = jnp.concatenate([W4, W8, W16, W32], axis=1)
    return _run(x, adj, wcat)


# speedup vs baseline: 1.3026x; 1.1025x over previous
import jax
import jax.numpy as jnp
from jax.experimental import pallas as pl

N = 4096
D = 512
BM = 1024


def _body(adj_ref, x_ref, w4_ref, w8_ref, w16_ref, w32_ref, o_ref):
    y = jnp.dot(adj_ref[...], x_ref[...], preferred_element_type=jnp.float32)
    acc = jnp.maximum(jnp.dot(y, w4_ref[...], preferred_element_type=jnp.float32), 0.0)
    acc += jnp.maximum(jnp.dot(y, w8_ref[...], preferred_element_type=jnp.float32), 0.0)
    acc += jnp.maximum(jnp.dot(y, w16_ref[...], preferred_element_type=jnp.float32), 0.0)
    acc += jnp.maximum(jnp.dot(y, w32_ref[...], preferred_element_type=jnp.float32), 0.0)
    o_ref[...] = acc


@jax.jit
def _run(x, adj, W4, W8, W16, W32):
    grid = (N // BM,)
    w_spec = pl.BlockSpec((D, D), lambda i: (0, 0))
    return pl.pallas_call(
        _body,
        grid=grid,
        in_specs=[
            pl.BlockSpec((BM, N), lambda i: (i, 0)),
            pl.BlockSpec((N, D), lambda i: (0, 0)),
            w_spec, w_spec, w_spec, w_spec,
        ],
        out_specs=pl.BlockSpec((BM, D), lambda i: (i, 0)),
        out_shape=jax.ShapeDtypeStruct((N, D), jnp.float32),
    )(adj, x, W4, W8, W16, W32)


def kernel(x, adj, now_epoch, W4, W8, W16, W32):
    return _run(x, adj, W4, W8, W16, W32)
